# baseline (speedup 1.0000x reference)
import functools

import jax
import jax.numpy as jnp
from jax import lax
from jax.experimental import pallas as pl
from jax.experimental.pallas import tpu as pltpu
from jax.experimental.pallas import tpu_sc as plsc

_B, _F, _V, _D, _C = 16384, 26, 100000, 16, 13
_H0, _H1 = 512, 256
_E = _F * _D
_EPS = 1e-5

_NC, _NS = 2, 16
_NW = _NC * _NS
_RPW = _E // _NW
_VA = 50048
_VB = _V - _VA
_IH = 8192


def _make_gather():
  mesh = plsc.VectorSubcoreMesh(core_axis_name="c", subcore_axis_name="s")

  @functools.partial(
      pl.kernel,
      mesh=mesh,
      out_type=jax.ShapeDtypeStruct((_E, _B), jnp.float32),
      scratch_types=[
          pltpu.VMEM((_VA,), jnp.float32),
          pltpu.VMEM((_VB,), jnp.float32),
          pltpu.VMEM((_IH,), jnp.int32),
          pltpu.VMEM((_B,), jnp.float32),
          pltpu.SemaphoreType.DMA,
          pltpu.SemaphoreType.DMA,
      ],
      compiler_params=pltpu.CompilerParams(use_tc_tiling_on_sc=True,
                                           needs_layout_passes=False),
  )
  def gather_k(m_hbm, idx_hbm, out_hbm, row_a, row_b, idx_v, out_v,
               sem_a, sem_b):
    wid = lax.axis_index("s") * _NC + lax.axis_index("c")
    r0 = wid * _RPW
    pltpu.async_copy(m_hbm.at[r0, pl.ds(0, _VA)], row_a, sem_a)

    def do_row(t, carry):
      r = r0 + t
      i = r // _D
      pltpu.make_async_copy(m_hbm.at[r, pl.ds(0, _VA)], row_a, sem_a).wait()
      pltpu.async_copy(m_hbm.at[r, pl.ds(_VA, _VB)], row_b, sem_b)

      for h in (0, 1):
        pltpu.sync_copy(idx_hbm.at[i, pl.ds(h * _IH, _IH)], idx_v)

        @plsc.parallel_loop(0, _IH // 16, unroll=8)
        def _(k):
          o = pl.multiple_of(k * 16, 16)
          v = idx_v[pl.ds(o, 16)]
          g = plsc.load_gather(row_a, [jnp.minimum(v, _VA - 1)], mask=v < _VA)
          out_v[pl.ds(h * _IH + o, 16)] = g

      pltpu.make_async_copy(m_hbm.at[r, pl.ds(_VA, _VB)], row_b, sem_b).wait()

      @pl.when(t + 1 < _RPW)
      def _():
        pltpu.async_copy(m_hbm.at[r + 1, pl.ds(0, _VA)], row_a, sem_a)

      for h in (0, 1):
        pltpu.sync_copy(idx_hbm.at[i, pl.ds(h * _IH, _IH)], idx_v)

        @plsc.parallel_loop(0, _IH // 16, unroll=8)
        def _(k):
          o = pl.multiple_of(k * 16, 16)
          v = idx_v[pl.ds(o, 16)]
          m = v >= _VA
          g = plsc.load_gather(row_b, [jnp.maximum(v - _VA, 0)], mask=m)
          cur = out_v[pl.ds(h * _IH + o, 16)]
          out_v[pl.ds(h * _IH + o, 16)] = jnp.where(m, g, cur)

      pltpu.sync_copy(out_v, out_hbm.at[r, :])
      return carry

    lax.fori_loop(0, _RPW, do_row, 0)

  return gather_k


_gather = _make_gather()


def _mlp_body(xcf_ref, emb_ref, xc_ref, w0a_ref, w0b_ref, b0_ref, w1_ref,
              b1_ref, wo_ref, bo_ref, out_ref, stats_ref):
  @pl.when(pl.program_id(0) == 0)
  def _():
    xc = xcf_ref[...]
    m = jnp.mean(xc, axis=1, keepdims=True)
    v = jnp.mean(xc * xc, axis=1, keepdims=True) - m * m
    stats_ref[:, 0:1] = m
    stats_ref[:, 1:2] = lax.rsqrt(v + _EPS)

  m = stats_ref[:, 0:1]
  rstd = stats_ref[:, 1:2]
  x2 = (xc_ref[...] - m) * rstd
  h = lax.dot_general(w0a_ref[...], emb_ref[...],
                      (((1,), (0,)), ((), ())),
                      preferred_element_type=jnp.float32)
  h = h + lax.dot_general(w0b_ref[...], x2,
                          (((1,), (0,)), ((), ())),
                          preferred_element_type=jnp.float32)
  h = jnp.maximum(h + b0_ref[...], 0.0)
  h = lax.dot_general(w1_ref[...], h,
                      (((1,), (0,)), ((), ())),
                      preferred_element_type=jnp.float32)
  h = jnp.maximum(h + b1_ref[...], 0.0)
  out_ref[...] = jnp.sum(h * wo_ref[...], axis=0, keepdims=True) + bo_ref[...]


_BLKN = 1024


def _mlp(embT, xcT, w0a, w0b_eff, b0c, w1, b1c, wo_c, bo_c):
  grid = (_B // _BLKN,)
  return pl.pallas_call(
      _mlp_body,
      grid=grid,
      in_specs=[
          pl.BlockSpec((_C, _B), lambda j: (0, 0)),
          pl.BlockSpec((_E, _BLKN), lambda j: (0, j)),
          pl.BlockSpec((_C, _BLKN), lambda j: (0, j)),
          pl.BlockSpec((_H0, _E), lambda j: (0, 0)),
          pl.BlockSpec((_H0, _C), lambda j: (0, 0)),
          pl.BlockSpec((_H0, 1), lambda j: (0, 0)),
          pl.BlockSpec((_H1, _H0), lambda j: (0, 0)),
          pl.BlockSpec((_H1, 1), lambda j: (0, 0)),
          pl.BlockSpec((_H1, 1), lambda j: (0, 0)),
          pl.BlockSpec((1, 1), lambda j: (0, 0)),
      ],
      out_specs=pl.BlockSpec((1, _BLKN), lambda j: (0, j)),
      out_shape=jax.ShapeDtypeStruct((1, _B), jnp.float32),
      scratch_shapes=[pltpu.VMEM((_C, 128), jnp.float32)],
  )(xcT, embT, xcT, w0a, w0b_eff, b0c, w1, b1c, wo_c, bo_c)


def kernel(x_cat, x_cont, tables, gamma, beta, W0, b0, W1, b1, Wout, bout):
  m_mat = jnp.transpose(tables, (0, 2, 1)).reshape(_E, _V)
  idxT = x_cat.T
  xcT = x_cont.T

  embT = _gather(m_mat, idxT)

  w0a = W0[:, :_E]
  w0b = W0[:, _E:]
  w0b_eff = w0b * gamma[None, :]
  b0c = (b0 + w0b @ beta).reshape(_H0, 1)
  b1c = b1.reshape(_H1, 1)
  wo_c = Wout.reshape(_H1, 1)
  bo_c = bout.reshape(1, 1)

  outT = _mlp(embT, xcT, w0a, w0b_eff, b0c, W1, b1c, wo_c, bo_c)
  return outT.reshape(_B, 1)


# speedup vs baseline: 1.1532x; 1.1532x over previous
import functools

import jax
import jax.numpy as jnp
from jax import lax
from jax.experimental import pallas as pl
from jax.experimental.pallas import tpu as pltpu
from jax.experimental.pallas import tpu_sc as plsc

_B, _F, _V, _D, _C = 16384, 26, 100000, 16, 13
_H0, _H1 = 512, 256
_E = _F * _D
_EPS = 1e-5

_NC, _NS = 2, 16
_NW = _NC * _NS
_RPW = _E // _NW
_VA = 50048
_VB = _V - _VA
_W = _B // 2


def _round_bf16_bits(g):
  u = plsc.bitcast(g, jnp.int32)
  lsb = lax.shift_right_logical(u, 16) & 1
  return lax.shift_right_logical(u + 0x7FFF + lsb, 16)


def _make_gather():
  mesh = plsc.VectorSubcoreMesh(core_axis_name="c", subcore_axis_name="s")

  @functools.partial(
      pl.kernel,
      mesh=mesh,
      out_type=jax.ShapeDtypeStruct((_E, _W), jnp.int32),
      scratch_types=[
          pltpu.VMEM((_VA,), jnp.float32),
          pltpu.VMEM((_VB,), jnp.float32),
          pltpu.VMEM((_B,), jnp.int32),
          pltpu.VMEM((_W,), jnp.int32),
          pltpu.SemaphoreType.DMA,
          pltpu.SemaphoreType.DMA,
          pltpu.SemaphoreType.DMA,
      ],
      compiler_params=pltpu.CompilerParams(use_tc_tiling_on_sc=True,
                                           needs_layout_passes=False),
  )
  def gather_k(m_hbm, idx_hbm, out_hbm, row_a, row_b, idx_v, out_v,
               sem_a, sem_b, sem_w):
    wid = lax.axis_index("s") * _NC + lax.axis_index("c")
    r0 = wid * _RPW
    pltpu.async_copy(m_hbm.at[r0, pl.ds(0, _VA)], row_a, sem_a)

    def do_row(t, prev_i):
      r = r0 + t
      i = r // _D
      pltpu.make_async_copy(m_hbm.at[r, pl.ds(0, _VA)], row_a, sem_a).wait()

      @pl.when(i != prev_i)
      def _():
        pltpu.sync_copy(idx_hbm.at[i, :], idx_v)

      pltpu.async_copy(m_hbm.at[r, pl.ds(_VA, _VB)], row_b, sem_b)

      @pl.when(t > 0)
      def _():
        pltpu.make_async_copy(out_v, out_hbm.at[r - 1, :], sem_w).wait()

      @plsc.parallel_loop(0, _W // 16, unroll=8)
      def _(g):
        o = pl.multiple_of(g * 32, 32)
        ve = idx_v[pl.ds(o, 16)]
        vo = idx_v[pl.ds(o + 16, 16)]
        ge = plsc.load_gather(row_a, [jnp.minimum(ve, _VA - 1)],
                              mask=ve < _VA)
        go = plsc.load_gather(row_a, [jnp.minimum(vo, _VA - 1)],
                              mask=vo < _VA)
        word = (_round_bf16_bits(ge) & 0xFFFF) | lax.shift_left(
            _round_bf16_bits(go), 16)
        out_v[pl.ds(pl.multiple_of(g * 16, 16), 16)] = word

      pltpu.make_async_copy(m_hbm.at[r, pl.ds(_VA, _VB)], row_b, sem_b).wait()

      @pl.when(t + 1 < _RPW)
      def _():
        pltpu.async_copy(m_hbm.at[r + 1, pl.ds(0, _VA)], row_a, sem_a)

      @plsc.parallel_loop(0, _W // 16, unroll=8)
      def _(g):
        o = pl.multiple_of(g * 32, 32)
        ve = idx_v[pl.ds(o, 16)]
        vo = idx_v[pl.ds(o + 16, 16)]
        me = ve >= _VA
        mo = vo >= _VA
        ge = plsc.load_gather(row_b, [jnp.maximum(ve - _VA, 0)], mask=me)
        go = plsc.load_gather(row_b, [jnp.maximum(vo - _VA, 0)], mask=mo)
        word = out_v[pl.ds(pl.multiple_of(g * 16, 16), 16)]
        low = jnp.where(me, _round_bf16_bits(ge), word & 0xFFFF)
        high = jnp.where(mo, _round_bf16_bits(go),
                         lax.shift_right_logical(word, 16))
        out_v[pl.ds(pl.multiple_of(g * 16, 16), 16)] = low | lax.shift_left(
            high, 16)

      pltpu.async_copy(out_v, out_hbm.at[r, :], sem_w)
      return i

    lax.fori_loop(0, _RPW, do_row, jnp.int32(-1))
    pltpu.make_async_copy(out_v, out_hbm.at[r0 + _RPW - 1, :], sem_w).wait()

  return gather_k


_gather = _make_gather()


def _mlp_body(xcf_ref, emb_ref, xc_ref, w0a_ref, w0b_ref, b0_ref, w1_ref,
              b1_ref, wo_ref, bo_ref, out_ref, stats_ref):
  @pl.when(pl.program_id(0) == 0)
  def _():
    xc = xcf_ref[...]
    m = jnp.mean(xc, axis=1, keepdims=True)
    v = jnp.mean(xc * xc, axis=1, keepdims=True) - m * m
    stats_ref[:, 0:1] = m
    stats_ref[:, 1:2] = lax.rsqrt(v + _EPS)

  m = stats_ref[:, 0:1]
  rstd = stats_ref[:, 1:2]
  x2 = (xc_ref[...] - m) * rstd
  w = emb_ref[...]
  even = lax.bitcast_convert_type(lax.shift_left(w, 16), jnp.float32)
  odd = lax.bitcast_convert_type(w & jnp.int32(-65536), jnp.float32)
  emb = jnp.concatenate([even, odd], axis=1)
  h = lax.dot_general(w0a_ref[...], emb,
                      (((1,), (0,)), ((), ())),
                      preferred_element_type=jnp.float32)
  h = h + lax.dot_general(w0b_ref[...], x2,
                          (((1,), (0,)), ((), ())),
                          preferred_element_type=jnp.float32)
  h = jnp.maximum(h + b0_ref[...], 0.0)
  h = lax.dot_general(w1_ref[...], h,
                      (((1,), (0,)), ((), ())),
                      preferred_element_type=jnp.float32)
  h = jnp.maximum(h + b1_ref[...], 0.0)
  out_ref[...] = jnp.sum(h * wo_ref[...], axis=0, keepdims=True) + bo_ref[...]


_BLKN = 1024


def _mlp(embW, xcT, xcT_d, w0a, w0b_eff, b0c, w1, b1c, wo_c, bo_c):
  grid = (_B // _BLKN,)
  return pl.pallas_call(
      _mlp_body,
      grid=grid,
      in_specs=[
          pl.BlockSpec((_C, _B), lambda j: (0, 0)),
          pl.BlockSpec((_E, _BLKN // 2), lambda j: (0, j)),
          pl.BlockSpec((_C, _BLKN), lambda j: (0, j)),
          pl.BlockSpec((_H0, _E), lambda j: (0, 0)),
          pl.BlockSpec((_H0, _C), lambda j: (0, 0)),
          pl.BlockSpec((_H0, 1), lambda j: (0, 0)),
          pl.BlockSpec((_H1, _H0), lambda j: (0, 0)),
          pl.BlockSpec((_H1, 1), lambda j: (0, 0)),
          pl.BlockSpec((_H1, 1), lambda j: (0, 0)),
          pl.BlockSpec((1, 1), lambda j: (0, 0)),
      ],
      out_specs=pl.BlockSpec((1, _BLKN), lambda j: (0, j)),
      out_shape=jax.ShapeDtypeStruct((1, _B), jnp.float32),
      scratch_shapes=[pltpu.VMEM((_C, 128), jnp.float32)],
  )(xcT, embW, xcT_d, w0a, w0b_eff, b0c, w1, b1c, wo_c, bo_c)


def _deint32(x):
  n, b = x.shape
  return (x.reshape(n, b // 32, 16, 2)
          .transpose(0, 1, 3, 2)
          .reshape(n, b))


def _deint1024(x):
  n, b = x.shape
  return (x.reshape(n, b // 1024, 512, 2)
          .transpose(0, 1, 3, 2)
          .reshape(n, b))


def kernel(x_cat, x_cont, tables, gamma, beta, W0, b0, W1, b1, Wout, bout):
  m_mat = jnp.transpose(tables, (0, 2, 1)).reshape(_E, _V)
  idxT_d = _deint32(x_cat.T)
  xcT = x_cont.T
  xcT_d = _deint1024(xcT)

  embW = _gather(m_mat, idxT_d)

  w0a = W0[:, :_E]
  w0b = W0[:, _E:]
  w0b_eff = w0b * gamma[None, :]
  b0c = (b0 + w0b @ beta).reshape(_H0, 1)
  b1c = b1.reshape(_H1, 1)
  wo_c = Wout.reshape(_H1, 1)
  bo_c = bout.reshape(1, 1)

  outT = _mlp(embW, xcT, xcT_d, w0a, w0b_eff, b0c, W1, b1c, wo_c, bo_c)
  return outT.reshape(_B // _BLKN, 2, _BLKN // 2).transpose(0, 2, 1).reshape(_B, 1)


# speedup vs baseline: 1.2960x; 1.1238x over previous
import functools

import jax
import jax.numpy as jnp
from jax import lax
from jax.experimental import pallas as pl
from jax.experimental.pallas import tpu as pltpu
from jax.experimental.pallas import tpu_sc as plsc

_B, _F, _V, _D, _C = 16384, 26, 100000, 16, 13
_H0, _H1 = 512, 256
_E = _F * _D
_EPS = 1e-5

_NC, _NS = 2, 16
_NW = _NC * _NS
_RPW = _E // _NW
_VA = 50048
_VB = _V - _VA
_W = _B // 2


def _round_bf16_bits(g):
  u = plsc.bitcast(g, jnp.int32)
  lsb = lax.shift_right_logical(u, 16) & 1
  return lax.shift_right_logical(u + 0x7FFF + lsb, 16)


def _make_gather():
  mesh = plsc.VectorSubcoreMesh(core_axis_name="c", subcore_axis_name="s")

  @functools.partial(
      pl.kernel,
      mesh=mesh,
      out_type=jax.ShapeDtypeStruct((_E, _W), jnp.int32),
      scratch_types=[
          pltpu.VMEM((_VA,), jnp.float32),
          pltpu.VMEM((_VB,), jnp.float32),
          pltpu.VMEM((_B,), jnp.int32),
          pltpu.VMEM((_W,), jnp.int32),
          pltpu.SemaphoreType.DMA,
          pltpu.SemaphoreType.DMA,
          pltpu.SemaphoreType.DMA,
      ],
      compiler_params=pltpu.CompilerParams(use_tc_tiling_on_sc=True,
                                           needs_layout_passes=False),
  )
  def gather_k(m_hbm, idx_hbm, out_hbm, row_a, row_b, idx_v, out_v,
               sem_a, sem_b, sem_w):
    wid = lax.axis_index("s") * _NC + lax.axis_index("c")
    r0 = wid * _RPW
    pltpu.async_copy(m_hbm.at[r0, pl.ds(0, _VA)], row_a, sem_a)

    def do_row(t, prev_i):
      r = r0 + t
      i = r // _D
      pltpu.make_async_copy(m_hbm.at[r, pl.ds(0, _VA)], row_a, sem_a).wait()

      @pl.when(i != prev_i)
      def _():
        pltpu.sync_copy(idx_hbm.at[i, :], idx_v)

      pltpu.async_copy(m_hbm.at[r, pl.ds(_VA, _VB)], row_b, sem_b)

      @pl.when(t > 0)
      def _():
        pltpu.make_async_copy(out_v, out_hbm.at[r - 1, :], sem_w).wait()

      @plsc.parallel_loop(0, _W // 16, unroll=8)
      def _(g):
        o = pl.multiple_of(g * 32, 32)
        ve = idx_v[pl.ds(o, 16)]
        vo = idx_v[pl.ds(o + 16, 16)]
        ge = plsc.load_gather(row_a, [jnp.minimum(ve, _VA - 1)],
                              mask=ve < _VA)
        go = plsc.load_gather(row_a, [jnp.minimum(vo, _VA - 1)],
                              mask=vo < _VA)
        word = (_round_bf16_bits(ge) & 0xFFFF) | lax.shift_left(
            _round_bf16_bits(go), 16)
        out_v[pl.ds(pl.multiple_of(g * 16, 16), 16)] = word

      pltpu.make_async_copy(m_hbm.at[r, pl.ds(_VA, _VB)], row_b, sem_b).wait()

      @pl.when(t + 1 < _RPW)
      def _():
        pltpu.async_copy(m_hbm.at[r + 1, pl.ds(0, _VA)], row_a, sem_a)

      @plsc.parallel_loop(0, _W // 16, unroll=8)
      def _(g):
        o = pl.multiple_of(g * 32, 32)
        ve = idx_v[pl.ds(o, 16)]
        vo = idx_v[pl.ds(o + 16, 16)]
        me = ve >= _VA
        mo = vo >= _VA
        ge = plsc.load_gather(row_b, [jnp.maximum(ve - _VA, 0)], mask=me)
        go = plsc.load_gather(row_b, [jnp.maximum(vo - _VA, 0)], mask=mo)
        word = out_v[pl.ds(pl.multiple_of(g * 16, 16), 16)]
        low = jnp.where(me, _round_bf16_bits(ge), word & 0xFFFF)
        high = jnp.where(mo, _round_bf16_bits(go),
                         lax.shift_right_logical(word, 16))
        out_v[pl.ds(pl.multiple_of(g * 16, 16), 16)] = low | lax.shift_left(
            high, 16)

      pltpu.async_copy(out_v, out_hbm.at[r, :], sem_w)
      return i

    lax.fori_loop(0, _RPW, do_row, jnp.int32(-1))
    pltpu.make_async_copy(out_v, out_hbm.at[r0 + _RPW - 1, :], sem_w).wait()

  return gather_k


_gather = _make_gather()


def _mlp_body(xcf_ref, emb_ref, xc_ref, w0a_ref, w0b_ref, b0_ref, w1_ref,
              b1_ref, wo_ref, bo_ref, out_ref, stats_ref):
  @pl.when(pl.program_id(0) == 0)
  def _():
    xc = xcf_ref[...]
    m = jnp.mean(xc, axis=1, keepdims=True)
    v = jnp.mean(xc * xc, axis=1, keepdims=True) - m * m
    stats_ref[:, 0:1] = m
    stats_ref[:, 1:2] = lax.rsqrt(v + _EPS)

  m = stats_ref[:, 0:1]
  rstd = stats_ref[:, 1:2]
  x2 = (xc_ref[...] - m) * rstd
  w = emb_ref[...]
  even = lax.bitcast_convert_type(lax.shift_left(w, 16), jnp.float32)
  odd = lax.bitcast_convert_type(w & jnp.int32(-65536), jnp.float32)
  emb = jnp.concatenate([even, odd], axis=1)
  h = lax.dot_general(w0a_ref[...], emb,
                      (((1,), (0,)), ((), ())),
                      preferred_element_type=jnp.float32)
  h = h + lax.dot_general(w0b_ref[...], x2,
                          (((1,), (0,)), ((), ())),
                          preferred_element_type=jnp.float32)
  h = jnp.maximum(h + b0_ref[...], 0.0)
  h = lax.dot_general(w1_ref[...], h,
                      (((1,), (0,)), ((), ())),
                      preferred_element_type=jnp.float32)
  h = jnp.maximum(h + b1_ref[...], 0.0)
  out_ref[...] = jnp.sum(h * wo_ref[...], axis=0, keepdims=True) + bo_ref[...]


_BLKN = 1024


def _mlp(embW, xcT, xcT_d, w0a, w0b_eff, b0c, w1, b1c, wo_c, bo_c):
  grid = (_B // _BLKN,)
  return pl.pallas_call(
      _mlp_body,
      grid=grid,
      in_specs=[
          pl.BlockSpec((_C, _B), lambda j: (0, 0)),
          pl.BlockSpec((_E, _BLKN // 2), lambda j: (0, j)),
          pl.BlockSpec((_C, _BLKN), lambda j: (0, j)),
          pl.BlockSpec((_H0, _E), lambda j: (0, 0)),
          pl.BlockSpec((_H0, _C), lambda j: (0, 0)),
          pl.BlockSpec((_H0, 1), lambda j: (0, 0)),
          pl.BlockSpec((_H1, _H0), lambda j: (0, 0)),
          pl.BlockSpec((_H1, 1), lambda j: (0, 0)),
          pl.BlockSpec((_H1, 1), lambda j: (0, 0)),
          pl.BlockSpec((1, 1), lambda j: (0, 0)),
      ],
      out_specs=pl.BlockSpec((1, _BLKN), lambda j: (0, j)),
      out_shape=jax.ShapeDtypeStruct((1, _B), jnp.float32),
      scratch_shapes=[pltpu.VMEM((_C, 128), jnp.float32)],
  )(xcT, embW, xcT_d, w0a, w0b_eff, b0c, w1, b1c, wo_c, bo_c)


def _perm1024(x):
  n, b = x.shape
  return (x.reshape(n, b // 1024, 32, 2, 16)
          .transpose(0, 1, 3, 2, 4)
          .reshape(n, b))


def kernel(x_cat, x_cont, tables, gamma, beta, W0, b0, W1, b1, Wout, bout):
  m_mat = jnp.transpose(tables, (0, 2, 1)).reshape(_E, _V)
  idxT = x_cat.T
  xcT = x_cont.T
  xcT_d = _perm1024(xcT)

  embW = _gather(m_mat, idxT)

  w0a = W0[:, :_E]
  w0b = W0[:, _E:]
  w0b_eff = w0b * gamma[None, :]
  b0c = (b0 + w0b @ beta).reshape(_H0, 1)
  b1c = b1.reshape(_H1, 1)
  wo_c = Wout.reshape(_H1, 1)
  bo_c = bout.reshape(1, 1)

  outT = _mlp(embW, xcT, xcT_d, w0a, w0b_eff, b0c, W1, b1c, wo_c, bo_c)
  return (outT.reshape(_B // _BLKN, 2, 32, 16)
          .transpose(0, 2, 1, 3)
          .reshape(_B, 1))


# speedup vs baseline: 1.2996x; 1.0028x over previous
import functools

import jax
import jax.numpy as jnp
from jax import lax
from jax.experimental import pallas as pl
from jax.experimental.pallas import tpu as pltpu
from jax.experimental.pallas import tpu_sc as plsc

_B, _F, _V, _D, _C = 16384, 26, 100000, 16, 13
_H0, _H1 = 512, 256
_E = _F * _D
_EPS = 1e-5

_NC, _NS = 2, 16
_NW = _NC * _NS
_RPW = _E // _NW
_VA = 50048
_VB = _V - _VA
_W = _B // 2


def _round_bf16_bits(g):
  u = plsc.bitcast(g, jnp.int32)
  lsb = lax.shift_right_logical(u, 16) & 1
  return lax.shift_right_logical(u + 0x7FFF + lsb, 16)


def _make_gather():
  mesh = plsc.VectorSubcoreMesh(core_axis_name="c", subcore_axis_name="s")

  @functools.partial(
      pl.kernel,
      mesh=mesh,
      out_type=jax.ShapeDtypeStruct((_E, _W), jnp.int32),
      scratch_types=[
          pltpu.VMEM((_VA,), jnp.float32),
          pltpu.VMEM((_VB,), jnp.float32),
          pltpu.VMEM((_B,), jnp.int32),
          pltpu.VMEM((_W,), jnp.int32),
          pltpu.SemaphoreType.DMA,
          pltpu.SemaphoreType.DMA,
          pltpu.SemaphoreType.DMA,
      ],
      compiler_params=pltpu.CompilerParams(use_tc_tiling_on_sc=True,
                                           needs_layout_passes=False),
  )
  def gather_k(m_hbm, idx_hbm, out_hbm, row_a, row_b, idx_v, out_v,
               sem_a, sem_b, sem_w):
    wid = lax.axis_index("s") * _NC + lax.axis_index("c")
    r0 = wid * _RPW
    pltpu.async_copy(m_hbm.at[r0, pl.ds(0, _VA)], row_a, sem_a)

    def do_row(t, prev_i):
      r = r0 + t
      i = r // _D
      pltpu.make_async_copy(m_hbm.at[r, pl.ds(0, _VA)], row_a, sem_a).wait()

      @pl.when(i != prev_i)
      def _():
        pltpu.sync_copy(idx_hbm.at[i, :], idx_v)

      pltpu.async_copy(m_hbm.at[r, pl.ds(_VA, _VB)], row_b, sem_b)

      @pl.when(t > 0)
      def _():
        pltpu.make_async_copy(out_v, out_hbm.at[r - 1, :], sem_w).wait()

      @plsc.parallel_loop(0, _W // 16, unroll=8)
      def _(g):
        o = pl.multiple_of(g * 32, 32)
        ve = idx_v[pl.ds(o, 16)]
        vo = idx_v[pl.ds(o + 16, 16)]
        ge = plsc.load_gather(row_a, [jnp.minimum(ve, _VA - 1)],
                              mask=ve < _VA)
        go = plsc.load_gather(row_a, [jnp.minimum(vo, _VA - 1)],
                              mask=vo < _VA)
        word = (_round_bf16_bits(ge) & 0xFFFF) | lax.shift_left(
            _round_bf16_bits(go), 16)
        out_v[pl.ds(pl.multiple_of(g * 16, 16), 16)] = word

      pltpu.make_async_copy(m_hbm.at[r, pl.ds(_VA, _VB)], row_b, sem_b).wait()

      @pl.when(t + 1 < _RPW)
      def _():
        pltpu.async_copy(m_hbm.at[r + 1, pl.ds(0, _VA)], row_a, sem_a)

      @plsc.parallel_loop(0, _W // 16, unroll=8)
      def _(g):
        o = pl.multiple_of(g * 32, 32)
        ve = idx_v[pl.ds(o, 16)]
        vo = idx_v[pl.ds(o + 16, 16)]
        me = ve >= _VA
        mo = vo >= _VA
        ge = plsc.load_gather(row_b, [jnp.maximum(ve - _VA, 0)], mask=me)
        go = plsc.load_gather(row_b, [jnp.maximum(vo - _VA, 0)], mask=mo)
        word = out_v[pl.ds(pl.multiple_of(g * 16, 16), 16)]
        low = jnp.where(me, _round_bf16_bits(ge), word & 0xFFFF)
        high = jnp.where(mo, _round_bf16_bits(go),
                         lax.shift_right_logical(word, 16))
        out_v[pl.ds(pl.multiple_of(g * 16, 16), 16)] = low | lax.shift_left(
            high, 16)

      pltpu.async_copy(out_v, out_hbm.at[r, :], sem_w)
      return i

    lax.fori_loop(0, _RPW, do_row, jnp.int32(-1))
    pltpu.make_async_copy(out_v, out_hbm.at[r0 + _RPW - 1, :], sem_w).wait()

  return gather_k


_gather = _make_gather()


def _mlp_body(xcf_ref, emb_ref, xc_ref, w0a_ref, w0b_ref, b0_ref, w1_ref,
              b1_ref, wo_ref, bo_ref, out_ref, stats_ref):
  @pl.when(pl.program_id(0) == 0)
  def _():
    xc = xcf_ref[...]
    m = jnp.mean(xc, axis=1, keepdims=True)
    v = jnp.mean(xc * xc, axis=1, keepdims=True) - m * m
    stats_ref[:, 0:1] = m
    stats_ref[:, 1:2] = lax.rsqrt(v + _EPS)

  m = stats_ref[:, 0:1]
  rstd = stats_ref[:, 1:2]
  x2 = (xc_ref[...] - m) * rstd
  w = emb_ref[...]
  even = lax.bitcast_convert_type(lax.shift_left(w, 16), jnp.float32)
  odd = lax.bitcast_convert_type(w & jnp.int32(-65536), jnp.float32)
  emb = jnp.concatenate([even, odd], axis=1).astype(jnp.bfloat16)
  h = lax.dot_general(w0a_ref[...], emb,
                      (((1,), (0,)), ((), ())),
                      preferred_element_type=jnp.float32)
  h = h + lax.dot_general(w0b_ref[...], x2,
                          (((1,), (0,)), ((), ())),
                          preferred_element_type=jnp.float32)
  h = jnp.maximum(h + b0_ref[...], 0.0)
  h = lax.dot_general(w1_ref[...], h,
                      (((1,), (0,)), ((), ())),
                      preferred_element_type=jnp.float32)
  h = jnp.maximum(h + b1_ref[...], 0.0)
  out_ref[...] = jnp.sum(h * wo_ref[...], axis=0, keepdims=True) + bo_ref[...]


_BLKN = 1024


def _mlp(embW, xcT, xcT_d, w0a, w0b_eff, b0c, w1, b1c, wo_c, bo_c):
  grid = (_B // _BLKN,)
  return pl.pallas_call(
      _mlp_body,
      grid=grid,
      in_specs=[
          pl.BlockSpec((_C, _B), lambda j: (0, 0)),
          pl.BlockSpec((_E, _BLKN // 2), lambda j: (0, j)),
          pl.BlockSpec((_C, _BLKN), lambda j: (0, j)),
          pl.BlockSpec((_H0, _E), lambda j: (0, 0)),
          pl.BlockSpec((_H0, _C), lambda j: (0, 0)),
          pl.BlockSpec((_H0, 1), lambda j: (0, 0)),
          pl.BlockSpec((_H1, _H0), lambda j: (0, 0)),
          pl.BlockSpec((_H1, 1), lambda j: (0, 0)),
          pl.BlockSpec((_H1, 1), lambda j: (0, 0)),
          pl.BlockSpec((1, 1), lambda j: (0, 0)),
      ],
      out_specs=pl.BlockSpec((1, _BLKN), lambda j: (0, j)),
      out_shape=jax.ShapeDtypeStruct((1, _B), jnp.float32),
      scratch_shapes=[pltpu.VMEM((_C, 128), jnp.float32)],
  )(xcT, embW, xcT_d, w0a, w0b_eff, b0c, w1, b1c, wo_c, bo_c)


def _perm1024(x):
  n, b = x.shape
  return (x.reshape(n, b // 1024, 32, 2, 16)
          .transpose(0, 1, 3, 2, 4)
          .reshape(n, b))


def kernel(x_cat, x_cont, tables, gamma, beta, W0, b0, W1, b1, Wout, bout):
  m_mat = jnp.transpose(tables, (0, 2, 1)).reshape(_E, _V)
  idxT = x_cat.T
  xcT = x_cont.T
  xcT_d = _perm1024(xcT)

  embW = _gather(m_mat, idxT)

  w0a = W0[:, :_E].astype(jnp.bfloat16)
  w0b = W0[:, _E:]
  w0b_eff = w0b * gamma[None, :]
  b0c = (b0 + w0b @ beta).reshape(_H0, 1)
  b1c = b1.reshape(_H1, 1)
  wo_c = Wout.reshape(_H1, 1)
  bo_c = bout.reshape(1, 1)

  outT = _mlp(embW, xcT, xcT_d, w0a, w0b_eff, b0c, W1, b1c, wo_c, bo_c)
  return (outT.reshape(_B // _BLKN, 2, 32, 16)
          .transpose(0, 2, 1, 3)
          .reshape(_B, 1))
